# Initial kernel scaffold; baseline (speedup 1.0000x reference)
import jax
import jax.numpy as jnp
from jax.experimental import pallas as pl


def kernel(x, edge_index, w1, lin1_w, lin1_b, w2, lin2_w, lin2_b):
    raise NotImplementedError("write your pallas kernel here")


# speedup vs baseline: 21.5743x; 21.5743x over previous
import functools

import jax
import jax.numpy as jnp
from jax import lax
from jax.experimental import pallas as pl
from jax.experimental.pallas import tpu as pltpu
from jax.experimental.pallas import tpu_sc as plsc

N = 10000
E = 160000
L = 5
NTILES = 16
K = 128
EPT = -(-(E // NTILES) // K) * K
CPT = EPT // K
EPAD = EPT * NTILES
NACC = N + NTILES
ZR = NACC // NTILES
OPT8 = (N // NTILES) // 8 * 8
OLAST_OFF = OPT8 * (NTILES - 1)
OLAST = N - OLAST_OFF


def _make_spmm5(ch):
    mesh = plsc.VectorSubcoreMesh(core_axis_name="c", subcore_axis_name="s")
    out_type = [jax.ShapeDtypeStruct((2 * N, ch), jnp.float32) for _ in range(L)]
    scratch = [
        pltpu.VMEM_SHARED((NACC, ch), jnp.float32),
        pltpu.VMEM((CPT, K), jnp.int32),
        pltpu.VMEM((CPT, K), jnp.int32),
        pltpu.VMEM((K, ch), jnp.float32),
    ]

    @functools.partial(pl.kernel, out_type=out_type, mesh=mesh,
                       scratch_types=scratch,
                       compiler_params=pltpu.CompilerParams(
                           use_tc_tiling_on_sc=False))
    def spmm5(z0, rows_hbm, cols_hbm, zrows_hbm,
              o1, o2, o3, o4, o5, acc, rbuf, cbuf, gbuf):
        c = lax.axis_index("c")
        s = lax.axis_index("s")
        pltpu.sync_copy(rows_hbm.at[s], rbuf)
        pltpu.sync_copy(cols_hbm.at[c * NTILES + s], cbuf)
        outs = [o1, o2, o3, o4, o5]
        srcs = [z0, o1, o2, o3, o4]
        for it in range(L):
            pltpu.sync_copy(zrows_hbm, acc.at[pl.ds(s * ZR, ZR)])
            plsc.subcore_barrier()
            src = srcs[it]

            @pl.loop(0, CPT)
            def _(j, src=src):
                pltpu.sync_copy(src.at[cbuf.at[j]], gbuf)
                pltpu.sync_copy(gbuf, acc.at[rbuf.at[j]], add=True)

            plsc.subcore_barrier()
            out = outs[it]

            @pl.when(s < NTILES - 1)
            def _(out=out):
                pltpu.sync_copy(acc.at[pl.ds(s * OPT8, OPT8)],
                                out.at[pl.ds(c * N + s * OPT8, OPT8)])

            @pl.when(s == NTILES - 1)
            def _(out=out):
                pltpu.sync_copy(acc.at[pl.ds(OLAST_OFF, OLAST)],
                                out.at[pl.ds(c * N + OLAST_OFF, OLAST)])

            plsc.subcore_barrier()

    return spmm5


_spmm5_16 = _make_spmm5(16)
_spmm5_64 = _make_spmm5(64)

_BN = 1000


def _tc_prep(x, s_mat, w1v, w2v):
    def body(x_ref, s_ref, w1_ref, w2_ref, xs_ref, dinv_ref):
        s_blk = s_ref[...]
        deg1 = jnp.dot(s_blk, w1_ref[...], preferred_element_type=jnp.float32)
        deg2 = jnp.dot(s_blk, w2_ref[...], preferred_element_type=jnp.float32)
        dinv1 = jnp.where(deg1 > 0, lax.rsqrt(deg1), 0.0)
        dinv2 = jnp.where(deg2 > 0, lax.rsqrt(deg2), 0.0)
        xs_ref[...] = x_ref[...] * dinv1
        dinv_ref[...] = jnp.concatenate(
            [dinv1, dinv2, jnp.zeros((dinv1.shape[0], 6), jnp.float32)], axis=1)

    grid = (N // _BN,)
    return pl.pallas_call(
        body,
        grid=grid,
        in_specs=[
            pl.BlockSpec((_BN, 128), lambda i: (i, 0)),
            pl.BlockSpec((_BN, 8), lambda i: (i, 0)),
            pl.BlockSpec((8, 1), lambda i: (0, 0)),
            pl.BlockSpec((8, 1), lambda i: (0, 0)),
        ],
        out_specs=[
            pl.BlockSpec((_BN, 128), lambda i: (i, 0)),
            pl.BlockSpec((_BN, 8), lambda i: (i, 0)),
        ],
        out_shape=[
            jax.ShapeDtypeStruct((N, 128), jnp.float32),
            jax.ShapeDtypeStruct((N, 8), jnp.float32),
        ],
    )(x, s_mat, w1v, w2v)


def _tc_fused(xs, z1, z2, z3, z4, z5, dinv, w1, lin1_w, lin1_b, lin2_w):
    def body(xs_ref, z1_ref, z2_ref, z3_ref, z4_ref, z5_ref, dinv_ref,
             w1_ref, w1w_ref, b1_ref, w2w_ref, out_ref):
        acc = w1_ref[0] * xs_ref[...]
        for i, zr in enumerate((z1_ref, z2_ref, z3_ref, z4_ref, z5_ref)):
            acc = acc + w1_ref[i + 1] * zr[...]
        agg = acc * dinv_ref[:, 0:1]
        t = jnp.dot(agg, w1w_ref[...], preferred_element_type=jnp.float32)
        t = jnp.maximum(t + b1_ref[...], 0.0)
        t = t * dinv_ref[:, 1:2]
        out_ref[...] = jnp.dot(t, w2w_ref[...],
                               preferred_element_type=jnp.float32)

    grid = (N // _BN,)
    zspec = pl.BlockSpec((_BN, 128), lambda i: (i, 0))
    return pl.pallas_call(
        body,
        grid=grid,
        in_specs=[
            zspec, zspec, zspec, zspec, zspec, zspec,
            pl.BlockSpec((_BN, 8), lambda i: (i, 0)),
            pl.BlockSpec(memory_space=pltpu.SMEM),
            pl.BlockSpec((128, 3200), lambda i: (0, 0)),
            pl.BlockSpec((1, 3200), lambda i: (0, 0)),
            pl.BlockSpec((3200, 32), lambda i: (0, 0)),
        ],
        out_specs=pl.BlockSpec((_BN, 32), lambda i: (i, 0)),
        out_shape=jax.ShapeDtypeStruct((N, 32), jnp.float32),
    )(xs, z1, z2, z3, z4, z5, dinv, w1, lin1_w, lin1_b, lin2_w)


def _tc_final(hw2, z1, z2, z3, z4, z5, dinv, w2, lin2_b):
    def body(h_ref, z1_ref, z2_ref, z3_ref, z4_ref, z5_ref, dinv_ref,
             w2_ref, b2_ref, out_ref):
        acc = w2_ref[0] * h_ref[...]
        for i, zr in enumerate((z1_ref, z2_ref, z3_ref, z4_ref, z5_ref)):
            acc = acc + w2_ref[i + 1] * zr[...]
        out_ref[...] = jnp.maximum(acc * dinv_ref[:, 1:2] + b2_ref[...], 0.0)

    grid = (N // _BN,)
    zspec = pl.BlockSpec((_BN, 32), lambda i: (i, 0))
    return pl.pallas_call(
        body,
        grid=grid,
        in_specs=[
            zspec, zspec, zspec, zspec, zspec, zspec,
            pl.BlockSpec((_BN, 8), lambda i: (i, 0)),
            pl.BlockSpec(memory_space=pltpu.SMEM),
            pl.BlockSpec((1, 32), lambda i: (0, 0)),
        ],
        out_specs=zspec,
        out_shape=jax.ShapeDtypeStruct((N, 32), jnp.float32),
    )(hw2, z1, z2, z3, z4, z5, dinv, w2, lin2_b)


def _split(a):
    ch = a.shape[1] // 2
    return jnp.concatenate([a[:, :ch], a[:, ch:]], axis=0)


def _join(a):
    return jnp.concatenate([a[:N], a[N:]], axis=1)


def kernel(x, edge_index, w1, lin1_w, lin1_b, w2, lin2_w, lin2_b):
    row = edge_index[0]
    col = edge_index[1]
    pad = EPAD - E
    rowp = jnp.concatenate(
        [row, jnp.full((pad,), N, jnp.int32)]).reshape(NTILES, CPT, K)
    colp = jnp.concatenate([col, jnp.zeros((pad,), jnp.int32)])
    cols2 = jnp.stack(
        [colp.reshape(NTILES, CPT, K),
         (colp + N).reshape(NTILES, CPT, K)]).reshape(2 * NTILES, CPT, K)
    zr16 = jnp.zeros((ZR, 16), jnp.float32)
    zr64 = jnp.zeros((ZR, 64), jnp.float32)

    ones0 = jnp.zeros((2 * N, 16), jnp.float32).at[:, 0].set(1.0)
    dz = _spmm5_16(ones0, rowp, cols2, zr16)
    s_mat = jnp.stack(
        [o[:N, 0] for o in dz]
        + [jnp.ones((N,), jnp.float32)] + [jnp.zeros((N,), jnp.float32)] * 2,
        axis=1)
    w1v = jnp.concatenate([w1[1:], w1[0:1], jnp.zeros((2,), jnp.float32)])
    w2v = jnp.concatenate([w2[1:], w2[0:1], jnp.zeros((2,), jnp.float32)])
    xs, dinv = _tc_prep(x, s_mat, w1v.reshape(8, 1), w2v.reshape(8, 1))

    z = _spmm5_64(_split(xs), rowp, cols2, zr64)
    hw2 = _tc_fused(xs, *[_join(zz) for zz in z], dinv,
                    w1, lin1_w, lin1_b.reshape(1, -1), lin2_w)

    z2 = _spmm5_16(_split(hw2), rowp, cols2, zr16)
    return _tc_final(hw2, *[_join(zz) for zz in z2], dinv,
                     w2, lin2_b.reshape(1, -1))


# speedup vs baseline: 25.9261x; 1.2017x over previous
import functools

import jax
import jax.numpy as jnp
from jax import lax
from jax.experimental import pallas as pl
from jax.experimental.pallas import tpu as pltpu
from jax.experimental.pallas import tpu_sc as plsc

N = 10000
E = 160000
L = 5
NTILES = 16
K = 128
NBUF = 4
EPT = -(-(E // NTILES) // (K * NBUF)) * K * NBUF
CPT = EPT // K
NGRP = CPT // NBUF
EPAD = EPT * NTILES
NACC = N + NTILES
ZR = NACC // NTILES
OPT8 = (N // NTILES) // 8 * 8
OLAST_OFF = OPT8 * (NTILES - 1)
OLAST = N - OLAST_OFF


def _make_spmm5(ch):
    mesh = plsc.VectorSubcoreMesh(core_axis_name="c", subcore_axis_name="s")
    out_type = [jax.ShapeDtypeStruct((2 * N, ch), jnp.float32) for _ in range(L)]
    scratch = [
        pltpu.VMEM_SHARED((NACC, ch), jnp.float32),
        pltpu.VMEM((CPT, K), jnp.int32),
        pltpu.VMEM((CPT, K), jnp.int32),
        [pltpu.VMEM((K, ch), jnp.float32) for _ in range(NBUF)],
        pltpu.SemaphoreType.DMA((NBUF,)),
        pltpu.SemaphoreType.DMA((NBUF,)),
    ]

    @functools.partial(pl.kernel, out_type=out_type, mesh=mesh,
                       scratch_types=scratch,
                       compiler_params=pltpu.CompilerParams(
                           use_tc_tiling_on_sc=False))
    def spmm5(z0, rows_hbm, cols_hbm, zrows_hbm,
              o1, o2, o3, o4, o5, acc, rbuf, cbuf, gbufs, gsem, ssem):
        c = lax.axis_index("c")
        s = lax.axis_index("s")
        pltpu.sync_copy(rows_hbm.at[s], rbuf)
        pltpu.sync_copy(cols_hbm.at[c * NTILES + s], cbuf)
        outs = [o1, o2, o3, o4, o5]
        srcs = [z0, o1, o2, o3, o4]
        dummy = zrows_hbm.at[pl.ds(0, K)]
        for it in range(L):
            pltpu.sync_copy(zrows_hbm, acc.at[pl.ds(s * ZR, ZR)])
            plsc.subcore_barrier()
            src = srcs[it]

            @pl.loop(0, NGRP)
            def _(g, src=src):
                gds = []
                for b in range(NBUF):
                    j = g * NBUF + b

                    @pl.when(g > 0)
                    def _(b=b):
                        pltpu.make_async_copy(dummy, gbufs[b],
                                              ssem.at[b]).wait()

                    gds.append(pltpu.async_copy(
                        src.at[cbuf.at[j]], gbufs[b], gsem.at[b]))
                for b in range(NBUF):
                    j = g * NBUF + b
                    gds[b].wait()
                    pltpu.async_copy(gbufs[b], acc.at[rbuf.at[j]],
                                     ssem.at[b], add=True)

            for b in range(NBUF):
                pltpu.make_async_copy(dummy, gbufs[b], ssem.at[b]).wait()
            plsc.subcore_barrier()
            out = outs[it]

            @pl.when(s < NTILES - 1)
            def _(out=out):
                pltpu.sync_copy(acc.at[pl.ds(s * OPT8, OPT8)],
                                out.at[pl.ds(c * N + s * OPT8, OPT8)])

            @pl.when(s == NTILES - 1)
            def _(out=out):
                pltpu.sync_copy(acc.at[pl.ds(OLAST_OFF, OLAST)],
                                out.at[pl.ds(c * N + OLAST_OFF, OLAST)])

            plsc.subcore_barrier()

    return spmm5


_spmm5_16 = _make_spmm5(16)
_spmm5_64 = _make_spmm5(64)

_BN = 1000


def _tc_prep(x, s_mat, w1v, w2v):
    def body(x_ref, s_ref, w1_ref, w2_ref, xs_ref, dinv_ref):
        s_blk = s_ref[...]
        deg1 = jnp.dot(s_blk, w1_ref[...], preferred_element_type=jnp.float32)
        deg2 = jnp.dot(s_blk, w2_ref[...], preferred_element_type=jnp.float32)
        dinv1 = jnp.where(deg1 > 0, lax.rsqrt(deg1), 0.0)
        dinv2 = jnp.where(deg2 > 0, lax.rsqrt(deg2), 0.0)
        xs_ref[...] = x_ref[...] * dinv1
        dinv_ref[...] = jnp.concatenate(
            [dinv1, dinv2, jnp.zeros((dinv1.shape[0], 6), jnp.float32)], axis=1)

    grid = (N // _BN,)
    return pl.pallas_call(
        body,
        grid=grid,
        in_specs=[
            pl.BlockSpec((_BN, 128), lambda i: (i, 0)),
            pl.BlockSpec((_BN, 8), lambda i: (i, 0)),
            pl.BlockSpec((8, 1), lambda i: (0, 0)),
            pl.BlockSpec((8, 1), lambda i: (0, 0)),
        ],
        out_specs=[
            pl.BlockSpec((_BN, 128), lambda i: (i, 0)),
            pl.BlockSpec((_BN, 8), lambda i: (i, 0)),
        ],
        out_shape=[
            jax.ShapeDtypeStruct((N, 128), jnp.float32),
            jax.ShapeDtypeStruct((N, 8), jnp.float32),
        ],
    )(x, s_mat, w1v, w2v)


def _tc_fused(xs, z1, z2, z3, z4, z5, dinv, w1, lin1_w, lin1_b, lin2_w):
    def body(xs_ref, z1_ref, z2_ref, z3_ref, z4_ref, z5_ref, dinv_ref,
             w1_ref, w1w_ref, b1_ref, w2w_ref, out_ref):
        acc = w1_ref[0] * xs_ref[...]
        for i, zr in enumerate((z1_ref, z2_ref, z3_ref, z4_ref, z5_ref)):
            acc = acc + w1_ref[i + 1] * zr[...]
        agg = acc * dinv_ref[:, 0:1]
        t = jnp.dot(agg, w1w_ref[...], preferred_element_type=jnp.float32)
        t = jnp.maximum(t + b1_ref[...], 0.0)
        t = t * dinv_ref[:, 1:2]
        out_ref[...] = jnp.dot(t, w2w_ref[...],
                               preferred_element_type=jnp.float32)

    grid = (N // _BN,)
    zspec = pl.BlockSpec((_BN, 128), lambda i: (i, 0))
    return pl.pallas_call(
        body,
        grid=grid,
        in_specs=[
            zspec, zspec, zspec, zspec, zspec, zspec,
            pl.BlockSpec((_BN, 8), lambda i: (i, 0)),
            pl.BlockSpec(memory_space=pltpu.SMEM),
            pl.BlockSpec((128, 3200), lambda i: (0, 0)),
            pl.BlockSpec((1, 3200), lambda i: (0, 0)),
            pl.BlockSpec((3200, 32), lambda i: (0, 0)),
        ],
        out_specs=pl.BlockSpec((_BN, 32), lambda i: (i, 0)),
        out_shape=jax.ShapeDtypeStruct((N, 32), jnp.float32),
    )(xs, z1, z2, z3, z4, z5, dinv, w1, lin1_w, lin1_b, lin2_w)


def _tc_final(hw2, z1, z2, z3, z4, z5, dinv, w2, lin2_b):
    def body(h_ref, z1_ref, z2_ref, z3_ref, z4_ref, z5_ref, dinv_ref,
             w2_ref, b2_ref, out_ref):
        acc = w2_ref[0] * h_ref[...]
        for i, zr in enumerate((z1_ref, z2_ref, z3_ref, z4_ref, z5_ref)):
            acc = acc + w2_ref[i + 1] * zr[...]
        out_ref[...] = jnp.maximum(acc * dinv_ref[:, 1:2] + b2_ref[...], 0.0)

    grid = (N // _BN,)
    zspec = pl.BlockSpec((_BN, 32), lambda i: (i, 0))
    return pl.pallas_call(
        body,
        grid=grid,
        in_specs=[
            zspec, zspec, zspec, zspec, zspec, zspec,
            pl.BlockSpec((_BN, 8), lambda i: (i, 0)),
            pl.BlockSpec(memory_space=pltpu.SMEM),
            pl.BlockSpec((1, 32), lambda i: (0, 0)),
        ],
        out_specs=zspec,
        out_shape=jax.ShapeDtypeStruct((N, 32), jnp.float32),
    )(hw2, z1, z2, z3, z4, z5, dinv, w2, lin2_b)


def _split(a):
    ch = a.shape[1] // 2
    return jnp.concatenate([a[:, :ch], a[:, ch:]], axis=0)


def _join(a):
    return jnp.concatenate([a[:N], a[N:]], axis=1)


def kernel(x, edge_index, w1, lin1_w, lin1_b, w2, lin2_w, lin2_b):
    row = edge_index[0]
    col = edge_index[1]
    pad = EPAD - E
    rowp = jnp.concatenate(
        [row, jnp.full((pad,), N, jnp.int32)]).reshape(NTILES, CPT, K)
    colp = jnp.concatenate([col, jnp.zeros((pad,), jnp.int32)])
    cols2 = jnp.stack(
        [colp.reshape(NTILES, CPT, K),
         (colp + N).reshape(NTILES, CPT, K)]).reshape(2 * NTILES, CPT, K)
    zr16 = jnp.zeros((ZR, 16), jnp.float32)
    zr64 = jnp.zeros((ZR, 64), jnp.float32)

    ones0 = jnp.zeros((2 * N, 16), jnp.float32).at[:, 0].set(1.0)
    dz = _spmm5_16(ones0, rowp, cols2, zr16)
    s_mat = jnp.stack(
        [o[:N, 0] for o in dz]
        + [jnp.ones((N,), jnp.float32)] + [jnp.zeros((N,), jnp.float32)] * 2,
        axis=1)
    w1v = jnp.concatenate([w1[1:], w1[0:1], jnp.zeros((2,), jnp.float32)])
    w2v = jnp.concatenate([w2[1:], w2[0:1], jnp.zeros((2,), jnp.float32)])
    xs, dinv = _tc_prep(x, s_mat, w1v.reshape(8, 1), w2v.reshape(8, 1))

    z = _spmm5_64(_split(xs), rowp, cols2, zr64)
    hw2 = _tc_fused(xs, *[_join(zz) for zz in z], dinv,
                    w1, lin1_w, lin1_b.reshape(1, -1), lin2_w)

    z2 = _spmm5_16(_split(hw2), rowp, cols2, zr16)
    return _tc_final(hw2, *[_join(zz) for zz in z2], dinv,
                     w2, lin2_b.reshape(1, -1))


# speedup vs baseline: 36.8117x; 1.4199x over previous
import functools

import jax
import jax.numpy as jnp
from jax import lax
from jax.experimental import pallas as pl
from jax.experimental.pallas import tpu as pltpu
from jax.experimental.pallas import tpu_sc as plsc

N = 10000
E = 160000
L = 5
NTILES = 16
K = 128
NBUF = 3
EPT = -(-(E // NTILES) // (K * NBUF)) * K * NBUF
CPT = EPT // K
NGRP = CPT // NBUF
EPAD = EPT * NTILES
NACC = N + NTILES
ZR = NACC // NTILES


def _make_spmm5(ch):
    mesh = plsc.VectorSubcoreMesh(core_axis_name="c", subcore_axis_name="s")
    out_type = [jax.ShapeDtypeStruct((2 * N, ch), jnp.float32) for _ in range(L)]
    scratch = [
        pltpu.VMEM_SHARED((NACC, ch), jnp.float32),
        pltpu.VMEM_SHARED((NACC, ch), jnp.float32),
        pltpu.VMEM((CPT, K), jnp.int32),
        pltpu.VMEM((CPT, K), jnp.int32),
        [pltpu.VMEM((K, ch), jnp.float32) for _ in range(NBUF)],
        pltpu.SemaphoreType.DMA((NBUF,)),
        pltpu.SemaphoreType.DMA((NBUF,)),
    ]
    PRE = N // NTILES

    @functools.partial(pl.kernel, out_type=out_type, mesh=mesh,
                       scratch_types=scratch,
                       compiler_params=pltpu.CompilerParams(
                           use_tc_tiling_on_sc=False))
    def spmm5(z0, rows_hbm, cols_hbm, zrows_hbm,
              o1, o2, o3, o4, o5, ta, tb, rbuf, cbuf, gbufs, gsem, ssem):
        c = lax.axis_index("c")
        s = lax.axis_index("s")
        pltpu.sync_copy(rows_hbm.at[s], rbuf)
        pltpu.sync_copy(cols_hbm.at[s], cbuf)
        pltpu.sync_copy(z0.at[pl.ds(c * N + s * PRE, PRE)],
                        ta.at[pl.ds(s * PRE, PRE)])
        outs = [o1, o2, o3, o4, o5]
        dummy = zrows_hbm.at[pl.ds(0, K)]
        for it in range(L):
            table = (ta, tb)[it % 2]
            acc = (tb, ta)[it % 2]
            pltpu.sync_copy(zrows_hbm, acc.at[pl.ds(s * ZR, ZR)])
            plsc.subcore_barrier()

            @pl.loop(0, NGRP)
            def _(g, table=table, acc=acc):
                gds = []
                for b in range(NBUF):
                    j = g * NBUF + b

                    @pl.when(g > 0)
                    def _(b=b):
                        pltpu.make_async_copy(dummy, gbufs[b],
                                              ssem.at[b]).wait()

                    gds.append(pltpu.async_copy(
                        table.at[cbuf.at[j]], gbufs[b], gsem.at[b]))
                for b in range(NBUF):
                    j = g * NBUF + b
                    gds[b].wait()
                    pltpu.async_copy(gbufs[b], acc.at[rbuf.at[j]],
                                     ssem.at[b], add=True)

            for b in range(NBUF):
                pltpu.make_async_copy(dummy, gbufs[b], ssem.at[b]).wait()
            plsc.subcore_barrier()
            pltpu.sync_copy(acc.at[pl.ds(s * PRE, PRE)],
                            outs[it].at[pl.ds(c * N + s * PRE, PRE)])

    return spmm5


_spmm5_16 = _make_spmm5(16)
_spmm5_64 = _make_spmm5(64)

_BN = 1000


def _tc_prep(x, s_mat, w1v, w2v):
    def body(x_ref, s_ref, w1_ref, w2_ref, xs_ref, dinv_ref):
        s_blk = s_ref[...]
        deg1 = jnp.dot(s_blk, w1_ref[...], preferred_element_type=jnp.float32)
        deg2 = jnp.dot(s_blk, w2_ref[...], preferred_element_type=jnp.float32)
        dinv1 = jnp.where(deg1 > 0, lax.rsqrt(deg1), 0.0)
        dinv2 = jnp.where(deg2 > 0, lax.rsqrt(deg2), 0.0)
        xs_ref[...] = x_ref[...] * dinv1
        dinv_ref[...] = jnp.concatenate(
            [dinv1, dinv2, jnp.zeros((dinv1.shape[0], 6), jnp.float32)], axis=1)

    grid = (N // _BN,)
    return pl.pallas_call(
        body,
        grid=grid,
        in_specs=[
            pl.BlockSpec((_BN, 128), lambda i: (i, 0)),
            pl.BlockSpec((_BN, 8), lambda i: (i, 0)),
            pl.BlockSpec((8, 1), lambda i: (0, 0)),
            pl.BlockSpec((8, 1), lambda i: (0, 0)),
        ],
        out_specs=[
            pl.BlockSpec((_BN, 128), lambda i: (i, 0)),
            pl.BlockSpec((_BN, 8), lambda i: (i, 0)),
        ],
        out_shape=[
            jax.ShapeDtypeStruct((N, 128), jnp.float32),
            jax.ShapeDtypeStruct((N, 8), jnp.float32),
        ],
    )(x, s_mat, w1v, w2v)


def _tc_fused(xs, z1, z2, z3, z4, z5, dinv, w1, lin1_w, lin1_b, lin2_w):
    def body(xs_ref, z1_ref, z2_ref, z3_ref, z4_ref, z5_ref, dinv_ref,
             w1_ref, w1w_ref, b1_ref, w2w_ref, out_ref):
        acc = w1_ref[0] * xs_ref[...]
        for i, zr in enumerate((z1_ref, z2_ref, z3_ref, z4_ref, z5_ref)):
            acc = acc + w1_ref[i + 1] * zr[...]
        agg = acc * dinv_ref[:, 0:1]
        t = jnp.dot(agg, w1w_ref[...], preferred_element_type=jnp.float32)
        t = jnp.maximum(t + b1_ref[...], 0.0)
        t = t * dinv_ref[:, 1:2]
        out_ref[...] = jnp.dot(t, w2w_ref[...],
                               preferred_element_type=jnp.float32)

    grid = (N // _BN,)
    zspec = pl.BlockSpec((_BN, 128), lambda i: (i, 0))
    return pl.pallas_call(
        body,
        grid=grid,
        in_specs=[
            zspec, zspec, zspec, zspec, zspec, zspec,
            pl.BlockSpec((_BN, 8), lambda i: (i, 0)),
            pl.BlockSpec(memory_space=pltpu.SMEM),
            pl.BlockSpec((128, 3200), lambda i: (0, 0)),
            pl.BlockSpec((1, 3200), lambda i: (0, 0)),
            pl.BlockSpec((3200, 32), lambda i: (0, 0)),
        ],
        out_specs=pl.BlockSpec((_BN, 32), lambda i: (i, 0)),
        out_shape=jax.ShapeDtypeStruct((N, 32), jnp.float32),
    )(xs, z1, z2, z3, z4, z5, dinv, w1, lin1_w, lin1_b, lin2_w)


def _tc_final(hw2, z1, z2, z3, z4, z5, dinv, w2, lin2_b):
    def body(h_ref, z1_ref, z2_ref, z3_ref, z4_ref, z5_ref, dinv_ref,
             w2_ref, b2_ref, out_ref):
        acc = w2_ref[0] * h_ref[...]
        for i, zr in enumerate((z1_ref, z2_ref, z3_ref, z4_ref, z5_ref)):
            acc = acc + w2_ref[i + 1] * zr[...]
        out_ref[...] = jnp.maximum(acc * dinv_ref[:, 1:2] + b2_ref[...], 0.0)

    grid = (N // _BN,)
    zspec = pl.BlockSpec((_BN, 32), lambda i: (i, 0))
    return pl.pallas_call(
        body,
        grid=grid,
        in_specs=[
            zspec, zspec, zspec, zspec, zspec, zspec,
            pl.BlockSpec((_BN, 8), lambda i: (i, 0)),
            pl.BlockSpec(memory_space=pltpu.SMEM),
            pl.BlockSpec((1, 32), lambda i: (0, 0)),
        ],
        out_specs=zspec,
        out_shape=jax.ShapeDtypeStruct((N, 32), jnp.float32),
    )(hw2, z1, z2, z3, z4, z5, dinv, w2, lin2_b)


def _split(a):
    ch = a.shape[1] // 2
    return jnp.concatenate([a[:, :ch], a[:, ch:]], axis=0)


def _join(a):
    return jnp.concatenate([a[:N], a[N:]], axis=1)


def kernel(x, edge_index, w1, lin1_w, lin1_b, w2, lin2_w, lin2_b):
    row = edge_index[0]
    col = edge_index[1]
    pad = EPAD - E
    rowp = jnp.concatenate(
        [row, jnp.full((pad,), N, jnp.int32)]).reshape(NTILES, CPT, K)
    cols2 = jnp.concatenate(
        [col, jnp.zeros((pad,), jnp.int32)]).reshape(NTILES, CPT, K)
    zr16 = jnp.zeros((ZR, 16), jnp.float32)
    zr64 = jnp.zeros((ZR, 64), jnp.float32)

    ones0 = jnp.zeros((2 * N, 16), jnp.float32).at[:, 0].set(1.0)
    dz = _spmm5_16(ones0, rowp, cols2, zr16)
    s_mat = jnp.stack(
        [o[:N, 0] for o in dz]
        + [jnp.ones((N,), jnp.float32)] + [jnp.zeros((N,), jnp.float32)] * 2,
        axis=1)
    w1v = jnp.concatenate([w1[1:], w1[0:1], jnp.zeros((2,), jnp.float32)])
    w2v = jnp.concatenate([w2[1:], w2[0:1], jnp.zeros((2,), jnp.float32)])
    xs, dinv = _tc_prep(x, s_mat, w1v.reshape(8, 1), w2v.reshape(8, 1))

    z = _spmm5_64(_split(xs), rowp, cols2, zr64)
    hw2 = _tc_fused(xs, *[_join(zz) for zz in z], dinv,
                    w1, lin1_w, lin1_b.reshape(1, -1), lin2_w)

    z2 = _spmm5_16(_split(hw2), rowp, cols2, zr16)
    return _tc_final(hw2, *[_join(zz) for zz in z2], dinv,
                     w2, lin2_b.reshape(1, -1))


# speedup vs baseline: 43.6065x; 1.1846x over previous
import functools

import jax
import jax.numpy as jnp
from jax import lax
from jax.experimental import pallas as pl
from jax.experimental.pallas import tpu as pltpu
from jax.experimental.pallas import tpu_sc as plsc

N = 10000
E = 160000
L = 5
NTILES = 16
K = 128
NBUF = 3
EPT = -(-(E // NTILES) // (K * NBUF)) * K * NBUF
CPT = EPT // K
NGRP = CPT // NBUF
EPAD = EPT * NTILES
NACC = N + NTILES
ZR = NACC // NTILES


def _make_spmm5(ch):
    mesh = plsc.VectorSubcoreMesh(core_axis_name="c", subcore_axis_name="s")
    out_type = [jax.ShapeDtypeStruct((2 * N, ch), jnp.float32) for _ in range(L)]
    scratch = [
        pltpu.VMEM_SHARED((NACC, ch), jnp.float32),
        pltpu.VMEM_SHARED((NACC, ch), jnp.float32),
        pltpu.VMEM((CPT, K), jnp.int32),
        pltpu.VMEM((CPT, K), jnp.int32),
        [pltpu.VMEM((K, ch), jnp.float32) for _ in range(NBUF)],
        pltpu.SemaphoreType.DMA((NBUF,)),
        pltpu.SemaphoreType.DMA((NBUF,)),
    ]
    PRE = N // NTILES

    @functools.partial(pl.kernel, out_type=out_type, mesh=mesh,
                       scratch_types=scratch,
                       compiler_params=pltpu.CompilerParams(
                           use_tc_tiling_on_sc=False))
    def spmm5(z0a, z0b, rows_hbm, cols_hbm, zrows_hbm,
              o1, o2, o3, o4, o5, ta, tb, rbuf, cbuf, gbufs, gsem, ssem):
        c = lax.axis_index("c")
        s = lax.axis_index("s")
        pltpu.sync_copy(rows_hbm.at[s], rbuf)
        pltpu.sync_copy(cols_hbm.at[s], cbuf)

        @pl.when(c == 0)
        def _():
            pltpu.sync_copy(z0a.at[pl.ds(s * PRE, PRE)],
                            ta.at[pl.ds(s * PRE, PRE)])

        @pl.when(c == 1)
        def _():
            pltpu.sync_copy(z0b.at[pl.ds(s * PRE, PRE)],
                            ta.at[pl.ds(s * PRE, PRE)])
        outs = [o1, o2, o3, o4, o5]
        dummy = zrows_hbm.at[pl.ds(0, K)]
        for it in range(L):
            table = (ta, tb)[it % 2]
            acc = (tb, ta)[it % 2]
            pltpu.sync_copy(zrows_hbm, acc.at[pl.ds(s * ZR, ZR)])
            plsc.subcore_barrier()

            @pl.loop(0, NGRP)
            def _(g, table=table, acc=acc):
                gds = []
                for b in range(NBUF):
                    j = g * NBUF + b

                    @pl.when(g > 0)
                    def _(b=b):
                        pltpu.make_async_copy(dummy, gbufs[b],
                                              ssem.at[b]).wait()

                    gds.append(pltpu.async_copy(
                        table.at[cbuf.at[j]], gbufs[b], gsem.at[b]))
                for b in range(NBUF):
                    j = g * NBUF + b
                    gds[b].wait()
                    pltpu.async_copy(gbufs[b], acc.at[rbuf.at[j]],
                                     ssem.at[b], add=True)

            for b in range(NBUF):
                pltpu.make_async_copy(dummy, gbufs[b], ssem.at[b]).wait()
            plsc.subcore_barrier()
            pltpu.sync_copy(acc.at[pl.ds(s * PRE, PRE)],
                            outs[it].at[pl.ds(c * N + s * PRE, PRE)])

    return spmm5


_spmm5_16 = _make_spmm5(16)
_spmm5_64 = _make_spmm5(64)

_BN = 1000


def _lo(i):
    return (i, 0)


def _hi(i):
    return (N // _BN + i, 0)


def _tc_prep(x, d1, d2, d3, d4, d5, w1, w2):
    def body(x_ref, d1_ref, d2_ref, d3_ref, d4_ref, d5_ref, w1_ref, w2_ref,
             xsa_ref, xsb_ref, dinv_ref):
        drefs = (d1_ref, d2_ref, d3_ref, d4_ref, d5_ref)
        deg1 = jnp.full((_BN, 1), 0.0, jnp.float32) + w1_ref[0]
        deg2 = jnp.full((_BN, 1), 0.0, jnp.float32) + w2_ref[0]
        for i, dr in enumerate(drefs):
            si = dr[:, 0:1]
            deg1 = deg1 + w1_ref[i + 1] * si
            deg2 = deg2 + w2_ref[i + 1] * si
        dinv1 = jnp.where(deg1 > 0, lax.rsqrt(deg1), 0.0)
        dinv2 = jnp.where(deg2 > 0, lax.rsqrt(deg2), 0.0)
        xs = x_ref[...] * dinv1
        xsa_ref[...] = xs[:, :64]
        xsb_ref[...] = xs[:, 64:]
        dinv_ref[...] = jnp.concatenate(
            [dinv1, dinv2, jnp.zeros((_BN, 6), jnp.float32)], axis=1)

    dspec = pl.BlockSpec((_BN, 16), _lo)
    return pl.pallas_call(
        body,
        grid=(N // _BN,),
        in_specs=[
            pl.BlockSpec((_BN, 128), _lo),
            dspec, dspec, dspec, dspec, dspec,
            pl.BlockSpec(memory_space=pltpu.SMEM),
            pl.BlockSpec(memory_space=pltpu.SMEM),
        ],
        out_specs=[
            pl.BlockSpec((_BN, 64), _lo),
            pl.BlockSpec((_BN, 64), _lo),
            pl.BlockSpec((_BN, 8), _lo),
        ],
        out_shape=[
            jax.ShapeDtypeStruct((N, 64), jnp.float32),
            jax.ShapeDtypeStruct((N, 64), jnp.float32),
            jax.ShapeDtypeStruct((N, 8), jnp.float32),
        ],
    )(x, d1, d2, d3, d4, d5, w1, w2)


def _tc_fused(xsa, xsb, z1, z2, z3, z4, z5, dinv, w1, lin1_w, lin1_b, lin2_w):
    def body(xsa_ref, xsb_ref,
             z1l, z1h, z2l, z2h, z3l, z3h, z4l, z4h, z5l, z5h,
             dinv_ref, w1_ref, w1w_ref, b1_ref, w2w_ref, ua_ref, ub_ref):
        lo = w1_ref[0] * xsa_ref[...]
        hi = w1_ref[0] * xsb_ref[...]
        for i, (zl, zh) in enumerate(
                ((z1l, z1h), (z2l, z2h), (z3l, z3h), (z4l, z4h), (z5l, z5h))):
            lo = lo + w1_ref[i + 1] * zl[...]
            hi = hi + w1_ref[i + 1] * zh[...]
        agg = jnp.concatenate([lo, hi], axis=1) * dinv_ref[:, 0:1]
        t = jnp.dot(agg, w1w_ref[...], preferred_element_type=jnp.float32)
        t = jnp.maximum(t + b1_ref[...], 0.0)
        t = t * dinv_ref[:, 1:2]
        u = jnp.dot(t, w2w_ref[...], preferred_element_type=jnp.float32)
        ua_ref[...] = u[:, :16]
        ub_ref[...] = u[:, 16:]

    zl = pl.BlockSpec((_BN, 64), _lo)
    zh = pl.BlockSpec((_BN, 64), _hi)
    return pl.pallas_call(
        body,
        grid=(N // _BN,),
        in_specs=[
            pl.BlockSpec((_BN, 64), _lo), pl.BlockSpec((_BN, 64), _lo),
            zl, zh, zl, zh, zl, zh, zl, zh, zl, zh,
            pl.BlockSpec((_BN, 8), _lo),
            pl.BlockSpec(memory_space=pltpu.SMEM),
            pl.BlockSpec((128, 3200), lambda i: (0, 0)),
            pl.BlockSpec((1, 3200), lambda i: (0, 0)),
            pl.BlockSpec((3200, 32), lambda i: (0, 0)),
        ],
        out_specs=[
            pl.BlockSpec((_BN, 16), _lo),
            pl.BlockSpec((_BN, 16), _lo),
        ],
        out_shape=[
            jax.ShapeDtypeStruct((N, 16), jnp.float32),
            jax.ShapeDtypeStruct((N, 16), jnp.float32),
        ],
    )(xsa, xsb, z1, z1, z2, z2, z3, z3, z4, z4, z5, z5,
      dinv, w1, lin1_w, lin1_b, lin2_w)


def _tc_final(ua, ub, z1, z2, z3, z4, z5, dinv, w2, lin2_b):
    def body(ua_ref, ub_ref,
             z1l, z1h, z2l, z2h, z3l, z3h, z4l, z4h, z5l, z5h,
             dinv_ref, w2_ref, b2_ref, out_ref):
        lo = w2_ref[0] * ua_ref[...]
        hi = w2_ref[0] * ub_ref[...]
        for i, (zl, zh) in enumerate(
                ((z1l, z1h), (z2l, z2h), (z3l, z3h), (z4l, z4h), (z5l, z5h))):
            lo = lo + w2_ref[i + 1] * zl[...]
            hi = hi + w2_ref[i + 1] * zh[...]
        acc = jnp.concatenate([lo, hi], axis=1)
        out_ref[...] = jnp.maximum(acc * dinv_ref[:, 1:2] + b2_ref[...], 0.0)

    zl = pl.BlockSpec((_BN, 16), _lo)
    zh = pl.BlockSpec((_BN, 16), _hi)
    return pl.pallas_call(
        body,
        grid=(N // _BN,),
        in_specs=[
            zl, zl,
            zl, zh, zl, zh, zl, zh, zl, zh, zl, zh,
            pl.BlockSpec((_BN, 8), _lo),
            pl.BlockSpec(memory_space=pltpu.SMEM),
            pl.BlockSpec((1, 32), lambda i: (0, 0)),
        ],
        out_specs=pl.BlockSpec((_BN, 32), _lo),
        out_shape=jax.ShapeDtypeStruct((N, 32), jnp.float32),
    )(ua, ub, z1, z1, z2, z2, z3, z3, z4, z4, z5, z5, dinv, w2, lin2_b)


def kernel(x, edge_index, w1, lin1_w, lin1_b, w2, lin2_w, lin2_b):
    row = edge_index[0]
    col = edge_index[1]
    pad = EPAD - E
    rowp = jnp.concatenate(
        [row, jnp.full((pad,), N, jnp.int32)]).reshape(NTILES, CPT, K)
    cols2 = jnp.concatenate(
        [col, jnp.zeros((pad,), jnp.int32)]).reshape(NTILES, CPT, K)
    zr16 = jnp.zeros((ZR, 16), jnp.float32)
    zr64 = jnp.zeros((ZR, 64), jnp.float32)

    ones0 = jnp.zeros((N, 16), jnp.float32).at[:, 0].set(1.0)
    dz = _spmm5_16(ones0, ones0, rowp, cols2, zr16)
    xsa, xsb, dinv = _tc_prep(x, *dz, w1, w2)

    z = _spmm5_64(xsa, xsb, rowp, cols2, zr64)
    ua, ub = _tc_fused(xsa, xsb, *z, dinv,
                       w1, lin1_w, lin1_b.reshape(1, -1), lin2_w)

    z2 = _spmm5_16(ua, ub, rowp, cols2, zr16)
    return _tc_final(ua, ub, *z2, dinv, w2, lin2_b.reshape(1, -1))


# speedup vs baseline: 43.9906x; 1.0088x over previous
import functools

import jax
import jax.numpy as jnp
from jax import lax
from jax.experimental import pallas as pl
from jax.experimental.pallas import tpu as pltpu
from jax.experimental.pallas import tpu_sc as plsc

N = 10000
E = 160000
L = 5
NTILES = 16
K = 128
EPT = -(-(E // NTILES) // (K * 3)) * K * 3
CPT = EPT // K
EPAD = EPT * NTILES
NACC = N + NTILES
ZR = NACC // NTILES


def _make_spmm5(ch, NBUF):
    assert CPT % NBUF == 0
    NGRP = CPT // NBUF
    mesh = plsc.VectorSubcoreMesh(core_axis_name="c", subcore_axis_name="s")
    out_type = [jax.ShapeDtypeStruct((2 * N, ch), jnp.float32) for _ in range(L)]
    scratch = [
        pltpu.VMEM_SHARED((NACC, ch), jnp.float32),
        pltpu.VMEM_SHARED((NACC, ch), jnp.float32),
        pltpu.VMEM((CPT, K), jnp.int32),
        pltpu.VMEM((CPT, K), jnp.int32),
        [pltpu.VMEM((K, ch), jnp.float32) for _ in range(NBUF)],
        pltpu.SemaphoreType.DMA((NBUF,)),
        pltpu.SemaphoreType.DMA((NBUF,)),
    ]
    PRE = N // NTILES

    @functools.partial(pl.kernel, out_type=out_type, mesh=mesh,
                       scratch_types=scratch,
                       compiler_params=pltpu.CompilerParams(
                           use_tc_tiling_on_sc=False))
    def spmm5(z0a, z0b, rows_hbm, cols_hbm, zrows_hbm,
              o1, o2, o3, o4, o5, ta, tb, rbuf, cbuf, gbufs, gsem, ssem):
        c = lax.axis_index("c")
        s = lax.axis_index("s")
        pltpu.sync_copy(rows_hbm.at[s], rbuf)
        pltpu.sync_copy(cols_hbm.at[s], cbuf)

        @pl.when(c == 0)
        def _():
            pltpu.sync_copy(z0a.at[pl.ds(s * PRE, PRE)],
                            ta.at[pl.ds(s * PRE, PRE)])

        @pl.when(c == 1)
        def _():
            pltpu.sync_copy(z0b.at[pl.ds(s * PRE, PRE)],
                            ta.at[pl.ds(s * PRE, PRE)])
        outs = [o1, o2, o3, o4, o5]
        dummy = zrows_hbm.at[pl.ds(0, K)]
        for it in range(L):
            table = (ta, tb)[it % 2]
            acc = (tb, ta)[it % 2]
            pltpu.sync_copy(zrows_hbm, acc.at[pl.ds(s * ZR, ZR)])
            plsc.subcore_barrier()

            @pl.loop(0, NGRP)
            def _(g, table=table, acc=acc):
                gds = []
                for b in range(NBUF):
                    j = g * NBUF + b

                    @pl.when(g > 0)
                    def _(b=b):
                        pltpu.make_async_copy(dummy, gbufs[b],
                                              ssem.at[b]).wait()

                    gds.append(pltpu.async_copy(
                        table.at[cbuf.at[j]], gbufs[b], gsem.at[b]))
                for b in range(NBUF):
                    j = g * NBUF + b
                    gds[b].wait()
                    pltpu.async_copy(gbufs[b], acc.at[rbuf.at[j]],
                                     ssem.at[b], add=True)

            for b in range(NBUF):
                pltpu.make_async_copy(dummy, gbufs[b], ssem.at[b]).wait()
            plsc.subcore_barrier()
            pltpu.sync_copy(acc.at[pl.ds(s * PRE, PRE)],
                            outs[it].at[pl.ds(c * N + s * PRE, PRE)])

    return spmm5


_spmm5_16 = _make_spmm5(16, 9)
_spmm5_64 = _make_spmm5(64, 3)

_BN = 1000


def _lo(i):
    return (i, 0)


def _hi(i):
    return (N // _BN + i, 0)


def _tc_prep(x, d1, d2, d3, d4, d5, w1, w2):
    def body(x_ref, d1_ref, d2_ref, d3_ref, d4_ref, d5_ref, w1_ref, w2_ref,
             xsa_ref, xsb_ref, dinv_ref):
        drefs = (d1_ref, d2_ref, d3_ref, d4_ref, d5_ref)
        deg1 = jnp.full((_BN, 1), 0.0, jnp.float32) + w1_ref[0]
        deg2 = jnp.full((_BN, 1), 0.0, jnp.float32) + w2_ref[0]
        for i, dr in enumerate(drefs):
            si = dr[:, 0:1]
            deg1 = deg1 + w1_ref[i + 1] * si
            deg2 = deg2 + w2_ref[i + 1] * si
        dinv1 = jnp.where(deg1 > 0, lax.rsqrt(deg1), 0.0)
        dinv2 = jnp.where(deg2 > 0, lax.rsqrt(deg2), 0.0)
        xs = x_ref[...] * dinv1
        xsa_ref[...] = xs[:, :64]
        xsb_ref[...] = xs[:, 64:]
        dinv_ref[...] = jnp.concatenate(
            [dinv1, dinv2, jnp.zeros((_BN, 6), jnp.float32)], axis=1)

    dspec = pl.BlockSpec((_BN, 16), _lo)
    return pl.pallas_call(
        body,
        grid=(N // _BN,),
        in_specs=[
            pl.BlockSpec((_BN, 128), _lo),
            dspec, dspec, dspec, dspec, dspec,
            pl.BlockSpec(memory_space=pltpu.SMEM),
            pl.BlockSpec(memory_space=pltpu.SMEM),
        ],
        out_specs=[
            pl.BlockSpec((_BN, 64), _lo),
            pl.BlockSpec((_BN, 64), _lo),
            pl.BlockSpec((_BN, 8), _lo),
        ],
        out_shape=[
            jax.ShapeDtypeStruct((N, 64), jnp.float32),
            jax.ShapeDtypeStruct((N, 64), jnp.float32),
            jax.ShapeDtypeStruct((N, 8), jnp.float32),
        ],
    )(x, d1, d2, d3, d4, d5, w1, w2)


def _tc_fused(xsa, xsb, z1, z2, z3, z4, z5, dinv, w1, lin1_w, lin1_b, lin2_w):
    def body(xsa_ref, xsb_ref,
             z1l, z1h, z2l, z2h, z3l, z3h, z4l, z4h, z5l, z5h,
             dinv_ref, w1_ref, w1w_ref, b1_ref, w2w_ref, ua_ref, ub_ref):
        lo = w1_ref[0] * xsa_ref[...]
        hi = w1_ref[0] * xsb_ref[...]
        for i, (zl, zh) in enumerate(
                ((z1l, z1h), (z2l, z2h), (z3l, z3h), (z4l, z4h), (z5l, z5h))):
            lo = lo + w1_ref[i + 1] * zl[...]
            hi = hi + w1_ref[i + 1] * zh[...]
        agg = jnp.concatenate([lo, hi], axis=1) * dinv_ref[:, 0:1]
        t = jnp.dot(agg, w1w_ref[...], preferred_element_type=jnp.float32)
        t = jnp.maximum(t + b1_ref[...], 0.0)
        t = t * dinv_ref[:, 1:2]
        u = jnp.dot(t, w2w_ref[...], preferred_element_type=jnp.float32)
        ua_ref[...] = u[:, :16]
        ub_ref[...] = u[:, 16:]

    zl = pl.BlockSpec((_BN, 64), _lo)
    zh = pl.BlockSpec((_BN, 64), _hi)
    return pl.pallas_call(
        body,
        grid=(N // _BN,),
        in_specs=[
            pl.BlockSpec((_BN, 64), _lo), pl.BlockSpec((_BN, 64), _lo),
            zl, zh, zl, zh, zl, zh, zl, zh, zl, zh,
            pl.BlockSpec((_BN, 8), _lo),
            pl.BlockSpec(memory_space=pltpu.SMEM),
            pl.BlockSpec((128, 3200), lambda i: (0, 0)),
            pl.BlockSpec((1, 3200), lambda i: (0, 0)),
            pl.BlockSpec((3200, 32), lambda i: (0, 0)),
        ],
        out_specs=[
            pl.BlockSpec((_BN, 16), _lo),
            pl.BlockSpec((_BN, 16), _lo),
        ],
        out_shape=[
            jax.ShapeDtypeStruct((N, 16), jnp.float32),
            jax.ShapeDtypeStruct((N, 16), jnp.float32),
        ],
    )(xsa, xsb, z1, z1, z2, z2, z3, z3, z4, z4, z5, z5,
      dinv, w1, lin1_w, lin1_b, lin2_w)


def _tc_final(ua, ub, z1, z2, z3, z4, z5, dinv, w2, lin2_b):
    def body(ua_ref, ub_ref,
             z1l, z1h, z2l, z2h, z3l, z3h, z4l, z4h, z5l, z5h,
             dinv_ref, w2_ref, b2_ref, out_ref):
        lo = w2_ref[0] * ua_ref[...]
        hi = w2_ref[0] * ub_ref[...]
        for i, (zl, zh) in enumerate(
                ((z1l, z1h), (z2l, z2h), (z3l, z3h), (z4l, z4h), (z5l, z5h))):
            lo = lo + w2_ref[i + 1] * zl[...]
            hi = hi + w2_ref[i + 1] * zh[...]
        acc = jnp.concatenate([lo, hi], axis=1)
        out_ref[...] = jnp.maximum(acc * dinv_ref[:, 1:2] + b2_ref[...], 0.0)

    zl = pl.BlockSpec((_BN, 16), _lo)
    zh = pl.BlockSpec((_BN, 16), _hi)
    return pl.pallas_call(
        body,
        grid=(N // _BN,),
        in_specs=[
            zl, zl,
            zl, zh, zl, zh, zl, zh, zl, zh, zl, zh,
            pl.BlockSpec((_BN, 8), _lo),
            pl.BlockSpec(memory_space=pltpu.SMEM),
            pl.BlockSpec((1, 32), lambda i: (0, 0)),
        ],
        out_specs=pl.BlockSpec((_BN, 32), _lo),
        out_shape=jax.ShapeDtypeStruct((N, 32), jnp.float32),
    )(ua, ub, z1, z1, z2, z2, z3, z3, z4, z4, z5, z5, dinv, w2, lin2_b)


def kernel(x, edge_index, w1, lin1_w, lin1_b, w2, lin2_w, lin2_b):
    row = edge_index[0]
    col = edge_index[1]
    pad = EPAD - E
    rowp = jnp.concatenate(
        [row, jnp.full((pad,), N, jnp.int32)]).reshape(NTILES, CPT, K)
    cols2 = jnp.concatenate(
        [col, jnp.zeros((pad,), jnp.int32)]).reshape(NTILES, CPT, K)
    zr16 = jnp.zeros((ZR, 16), jnp.float32)
    zr64 = jnp.zeros((ZR, 64), jnp.float32)

    ones0 = jnp.zeros((N, 16), jnp.float32).at[:, 0].set(1.0)
    dz = _spmm5_16(ones0, ones0, rowp, cols2, zr16)
    xsa, xsb, dinv = _tc_prep(x, *dz, w1, w2)

    z = _spmm5_64(xsa, xsb, rowp, cols2, zr64)
    ua, ub = _tc_fused(xsa, xsb, *z, dinv,
                       w1, lin1_w, lin1_b.reshape(1, -1), lin2_w)

    z2 = _spmm5_16(ua, ub, rowp, cols2, zr16)
    return _tc_final(ua, ub, *z2, dinv, w2, lin2_b.reshape(1, -1))
